# baseline (speedup 1.0000x reference)
import functools

import jax
import jax.numpy as jnp
from jax import lax
from jax.experimental import pallas as pl
from jax.experimental.pallas import tpu as pltpu
from jax.experimental.pallas import tpu_sc as plsc

B, C, H, W = 8, 3, 512, 512
HW = H * W
NC, NS, L = 2, 16, 16
NW = NC * NS
CHUNK = HW // NW
NVEC = CHUNK // L
NQ = 8


def _sc_partials(predf, tgtf, interpret=False):
    mesh = plsc.VectorSubcoreMesh(
        core_axis_name="c", subcore_axis_name="s",
        num_cores=NC, num_subcores=NS,
    )

    @functools.partial(
        pl.kernel,
        out_type=jax.ShapeDtypeStruct((NW * B * NQ * L,), jnp.float32),
        mesh=mesh,
        scratch_types=[
            pltpu.VMEM((CHUNK,), jnp.float32),
            pltpu.VMEM((CHUNK,), jnp.float32),
            pltpu.VMEM((CHUNK,), jnp.float32),
            pltpu.VMEM((CHUNK,), jnp.float32),
            pltpu.VMEM((CHUNK,), jnp.float32),
            pltpu.VMEM((CHUNK,), jnp.float32),
            pltpu.VMEM((B * NQ * L,), jnp.float32),
        ],
        interpret=interpret,
    )
    def k(pred_hbm, tgt_hbm, out_hbm, tv0, tv1, tv2, pv0, pv1, pv2, ov):
        wid = lax.axis_index("c") * NS + lax.axis_index("s")
        off = wid * CHUNK
        zero = jnp.zeros((L,), jnp.float32)
        one = jnp.ones((L,), jnp.float32)
        tvs = (tv0, tv1, tv2)
        pvs = (pv0, pv1, pv2)
        for b in range(B):
            for c in range(C):
                base = (b * C + c) * HW
                pltpu.sync_copy(tgt_hbm.at[pl.ds(base + off, CHUNK)], tvs[c])
                pltpu.sync_copy(pred_hbm.at[pl.ds(base + off, CHUNK)], pvs[c])

            def body(i, acc):
                s = pl.ds(i * L, L)
                t0 = tv0[s]
                t1 = tv1[s]
                t2 = tv2[s]
                p0 = pv0[s]
                p1 = pv1[s]
                p2 = pv2[s]
                bg = (t0 == 0.0) & (t1 == 0.0) & (t2 == 0.0)
                fg = (t0 == 255.0) & (t1 == 255.0) & (t2 == 255.0)
                return (
                    acc[0] + jnp.where(bg, one, zero),
                    acc[1] + jnp.where(fg, one, zero),
                    acc[2] + jnp.where(bg, p0, zero),
                    acc[3] + jnp.where(bg, p1, zero),
                    acc[4] + jnp.where(bg, p2, zero),
                    acc[5] + jnp.where(fg, p0, zero),
                    acc[6] + jnp.where(fg, p1, zero),
                    acc[7] + jnp.where(fg, p2, zero),
                )

            acc = lax.fori_loop(0, NVEC, body, (zero,) * NQ)
            for q in range(NQ):
                ov[pl.ds((b * NQ + q) * L, L)] = acc[q]
        pltpu.sync_copy(ov, out_hbm.at[pl.ds(wid * B * NQ * L, B * NQ * L)])

    return k(predf, tgtf)


def _combine(partials, interpret=False):
    def ck(p_ref, o_ref):
        x = p_ref[...]
        s = jnp.sum(x, axis=-1)
        bg_cnt = s[:, 0:1]
        fg_cnt = s[:, 1:2]
        bg_sum = s[:, 2:5]
        fg_sum = s[:, 5:8]
        has_bg = bg_cnt > 0.0
        has_fg = fg_cnt > 0.0
        bg_den = jnp.where(has_bg, bg_cnt, 1.0)
        fg_den = jnp.where(has_fg, fg_cnt, 1.0)
        bg_pred = bg_sum / bg_den
        fg_pred = fg_sum / fg_den

        def huber_mean(d):
            ad = jnp.abs(d)
            e = jnp.where(ad < 1.0, 0.5 * d * d, ad - 0.5)
            return jnp.mean(e, axis=1, keepdims=True)

        h_bg = huber_mean(bg_pred - 0.0)
        h_fg = huber_mean(fg_pred - 255.0)
        dist = jnp.sum((bg_pred - fg_pred) ** 2, axis=1, keepdims=True)
        both = jnp.logical_and(has_bg, has_fg)
        loss = (
            jnp.where(has_bg, h_bg, 0.0)
            + jnp.where(has_fg, h_fg, 0.0)
            + jnp.where(both, 300.0 / (1.0 + dist), 0.0)
        )
        valid = (
            has_bg.astype(jnp.float32)
            + has_fg.astype(jnp.float32)
            + both.astype(jnp.float32)
        )
        valid_den = jnp.where(valid > 0.0, valid, 1.0)
        per_b = jnp.where(valid > 0.0, loss / valid_den, 0.0)
        o_ref[...] = jnp.sum(per_b, keepdims=True) / B

    return pl.pallas_call(
        ck,
        out_shape=jax.ShapeDtypeStruct((1, 1), jnp.float32),
        interpret=interpret,
    )(partials)


def kernel(prediction, target):
    predf = prediction.astype(jnp.float32).reshape(B * C * HW)
    tgtf = target.astype(jnp.float32).reshape(B * C * HW)
    partials = _sc_partials(predf, tgtf)
    partials = partials.reshape(NW, B, NQ, L).transpose(1, 2, 0, 3)
    partials = partials.reshape(B, NQ, NW * L)
    total = _combine(partials)
    return total[0, 0]


# speedup vs baseline: 1.9578x; 1.9578x over previous
import functools

import jax
import jax.numpy as jnp
from jax import lax
from jax.experimental import pallas as pl
from jax.experimental.pallas import tpu as pltpu
from jax.experimental.pallas import tpu_sc as plsc

B, C, H, W = 8, 3, 512, 512
NC, NS, L = 2, 16, 16
NW = NC * NS
RPW = H // NW
NVEC = RPW * W // L
UNROLL = 4
NQ = 8


def _sc_partials(pred, tgt, interpret=False):
    mesh = plsc.VectorSubcoreMesh(
        core_axis_name="c", subcore_axis_name="s",
        num_cores=NC, num_subcores=NS,
    )

    slab = pltpu.VMEM((RPW, W), jnp.float32)

    @functools.partial(
        pl.kernel,
        out_type=jax.ShapeDtypeStruct((NW * B * NQ * L,), jnp.float32),
        mesh=mesh,
        scratch_types=[slab] * 12 + [
            pltpu.VMEM((B * NQ * L,), jnp.float32),
            pltpu.SemaphoreType.DMA,
            pltpu.SemaphoreType.DMA,
        ],
    )
    def k(pred_hbm, tgt_hbm, out_hbm, *refs):
        bufs = refs[:12]
        ov = refs[12]
        sems = refs[13:15]
        wid = lax.axis_index("c") * NS + lax.axis_index("s")
        r0 = wid * RPW
        zero = jnp.zeros((L,), jnp.float32)
        one = jnp.ones((L,), jnp.float32)

        def start(b, s):
            hs = []
            for c in range(C):
                hs.append(pltpu.async_copy(
                    tgt_hbm.at[b, c, pl.ds(r0, RPW), :], bufs[6 * s + c],
                    sems[s]))
                hs.append(pltpu.async_copy(
                    pred_hbm.at[b, c, pl.ds(r0, RPW), :], bufs[6 * s + 3 + c],
                    sems[s]))
            return hs

        pending = {0: start(0, 0)}
        for b in range(B):
            s = b & 1
            if b + 1 < B:
                pending[b + 1] = start(b + 1, (b + 1) & 1)
            for h in pending.pop(b):
                h.wait()
            tb0, tb1, tb2 = bufs[6 * s], bufs[6 * s + 1], bufs[6 * s + 2]
            pb0, pb1, pb2 = bufs[6 * s + 3], bufs[6 * s + 4], bufs[6 * s + 5]

            def body(i, acc):
                r = lax.shift_right_logical(i, 3)
                cb = pl.multiple_of(lax.shift_left(i & 7, 6), 64)
                for u in range(UNROLL):
                    sl = pl.ds(cb + u * L, L)
                    t0 = tb0[r, sl]
                    t1 = tb1[r, sl]
                    t2 = tb2[r, sl]
                    p0 = pb0[r, sl]
                    p1 = pb1[r, sl]
                    p2 = pb2[r, sl]
                    bg = (t0 == 0.0) & (t1 == 0.0) & (t2 == 0.0)
                    fg = (t0 == 255.0) & (t1 == 255.0) & (t2 == 255.0)
                    acc = (
                        acc[0] + jnp.where(bg, one, zero),
                        acc[1] + jnp.where(fg, one, zero),
                        acc[2] + jnp.where(bg, p0, zero),
                        acc[3] + jnp.where(bg, p1, zero),
                        acc[4] + jnp.where(bg, p2, zero),
                        acc[5] + jnp.where(fg, p0, zero),
                        acc[6] + jnp.where(fg, p1, zero),
                        acc[7] + jnp.where(fg, p2, zero),
                    )
                return acc

            acc = lax.fori_loop(0, NVEC // UNROLL, body, (zero,) * NQ)
            for q in range(NQ):
                ov[pl.ds((b * NQ + q) * L, L)] = acc[q]
        pltpu.sync_copy(ov, out_hbm.at[pl.ds(wid * B * NQ * L, B * NQ * L)])

    return k(pred, tgt)


def _combine(partials, interpret=False):
    def ck(p_ref, o_ref):
        x = p_ref[...]
        s = jnp.sum(x, axis=-1)
        bg_cnt = s[:, 0:1]
        fg_cnt = s[:, 1:2]
        bg_sum = s[:, 2:5]
        fg_sum = s[:, 5:8]
        has_bg = bg_cnt > 0.0
        has_fg = fg_cnt > 0.0
        bg_den = jnp.where(has_bg, bg_cnt, 1.0)
        fg_den = jnp.where(has_fg, fg_cnt, 1.0)
        bg_pred = bg_sum / bg_den
        fg_pred = fg_sum / fg_den

        def huber_mean(d):
            ad = jnp.abs(d)
            e = jnp.where(ad < 1.0, 0.5 * d * d, ad - 0.5)
            return jnp.mean(e, axis=1, keepdims=True)

        h_bg = huber_mean(bg_pred - 0.0)
        h_fg = huber_mean(fg_pred - 255.0)
        dist = jnp.sum((bg_pred - fg_pred) ** 2, axis=1, keepdims=True)
        both = jnp.logical_and(has_bg, has_fg)
        loss = (
            jnp.where(has_bg, h_bg, 0.0)
            + jnp.where(has_fg, h_fg, 0.0)
            + jnp.where(both, 300.0 / (1.0 + dist), 0.0)
        )
        valid = (
            has_bg.astype(jnp.float32)
            + has_fg.astype(jnp.float32)
            + both.astype(jnp.float32)
        )
        valid_den = jnp.where(valid > 0.0, valid, 1.0)
        per_b = jnp.where(valid > 0.0, loss / valid_den, 0.0)
        o_ref[...] = jnp.sum(per_b, keepdims=True) / B

    return pl.pallas_call(
        ck,
        out_shape=jax.ShapeDtypeStruct((1, 1), jnp.float32),
        interpret=interpret,
    )(partials)


def kernel(prediction, target):
    predf = prediction.astype(jnp.float32)
    tgtf = target.astype(jnp.float32)
    partials = _sc_partials(predf, tgtf)
    partials = partials.reshape(NW, B, NQ, L).transpose(1, 2, 0, 3)
    partials = partials.reshape(B, NQ, NW * L)
    total = _combine(partials)
    return total[0, 0]


# speedup vs baseline: 2.6616x; 1.3594x over previous
import functools

import jax
import jax.numpy as jnp
from jax import lax
from jax.experimental import pallas as pl
from jax.experimental.pallas import tpu as pltpu
from jax.experimental.pallas import tpu_sc as plsc

B, C, H, W = 8, 3, 512, 512
NC, NS, L = 2, 16, 16
NW = NC * NS
RPW = H // NW
NVEC = RPW * W // L
UNROLL = 4
NQ = 8


def _sc_partials(pred, tgt, interpret=False):
    mesh = plsc.VectorSubcoreMesh(
        core_axis_name="c", subcore_axis_name="s",
        num_cores=NC, num_subcores=NS,
    )

    slab = pltpu.VMEM((RPW, W), jnp.float32)

    @functools.partial(
        pl.kernel,
        out_type=jax.ShapeDtypeStruct((NW * B * NQ * L,), jnp.float32),
        mesh=mesh,
        scratch_types=[slab] * 12 + [
            pltpu.VMEM((B * NQ * L,), jnp.float32),
            pltpu.SemaphoreType.DMA,
            pltpu.SemaphoreType.DMA,
        ],
    )
    def k(pred_hbm, tgt_hbm, out_hbm, *refs):
        bufs = refs[:12]
        ov = refs[12]
        sems = refs[13:15]
        wid = lax.axis_index("c") * NS + lax.axis_index("s")
        r0 = wid * RPW
        zero = jnp.zeros((L,), jnp.float32)
        one = jnp.ones((L,), jnp.float32)

        def start(b, s):
            hs = []
            for c in range(C):
                hs.append(pltpu.async_copy(
                    tgt_hbm.at[b, c, pl.ds(r0, RPW), :], bufs[6 * s + c],
                    sems[s]))
                hs.append(pltpu.async_copy(
                    pred_hbm.at[b, c, pl.ds(r0, RPW), :], bufs[6 * s + 3 + c],
                    sems[s]))
            return hs

        pending = {0: start(0, 0)}
        for b in range(B):
            s = b & 1
            if b + 1 < B:
                pending[b + 1] = start(b + 1, (b + 1) & 1)
            for h in pending.pop(b):
                h.wait()
            tb0, tb1, tb2 = bufs[6 * s], bufs[6 * s + 1], bufs[6 * s + 2]
            pb0, pb1, pb2 = bufs[6 * s + 3], bufs[6 * s + 4], bufs[6 * s + 5]

            @plsc.parallel_loop(0, NVEC, unroll=UNROLL, carry=(zero,) * NQ)
            def acc(i, acc):
                r = lax.shift_right_logical(i, 5)
                cb = pl.multiple_of(lax.shift_left(i & 31, 4), 16)
                sl = pl.ds(cb, L)
                t0 = tb0[r, sl]
                t1 = tb1[r, sl]
                t2 = tb2[r, sl]
                p0 = pb0[r, sl]
                p1 = pb1[r, sl]
                p2 = pb2[r, sl]
                mbg = jnp.where(
                    (t0 == 0.0) & (t1 == 0.0) & (t2 == 0.0), one, zero)
                mfg = jnp.where(
                    (t0 == 255.0) & (t1 == 255.0) & (t2 == 255.0), one, zero)
                return (
                    acc[0] + mbg,
                    acc[1] + mfg,
                    acc[2] + mbg * p0,
                    acc[3] + mbg * p1,
                    acc[4] + mbg * p2,
                    acc[5] + mfg * p0,
                    acc[6] + mfg * p1,
                    acc[7] + mfg * p2,
                )
            for q in range(NQ):
                ov[pl.ds((b * NQ + q) * L, L)] = acc[q]
        pltpu.sync_copy(ov, out_hbm.at[pl.ds(wid * B * NQ * L, B * NQ * L)])

    return k(pred, tgt)


def _combine(partials, interpret=False):
    def ck(p_ref, o_ref):
        x = p_ref[...]
        s = jnp.sum(x, axis=-1)
        bg_cnt = s[:, 0:1]
        fg_cnt = s[:, 1:2]
        bg_sum = s[:, 2:5]
        fg_sum = s[:, 5:8]
        has_bg = bg_cnt > 0.0
        has_fg = fg_cnt > 0.0
        bg_den = jnp.where(has_bg, bg_cnt, 1.0)
        fg_den = jnp.where(has_fg, fg_cnt, 1.0)
        bg_pred = bg_sum / bg_den
        fg_pred = fg_sum / fg_den

        def huber_mean(d):
            ad = jnp.abs(d)
            e = jnp.where(ad < 1.0, 0.5 * d * d, ad - 0.5)
            return jnp.mean(e, axis=1, keepdims=True)

        h_bg = huber_mean(bg_pred - 0.0)
        h_fg = huber_mean(fg_pred - 255.0)
        dist = jnp.sum((bg_pred - fg_pred) ** 2, axis=1, keepdims=True)
        both = jnp.logical_and(has_bg, has_fg)
        loss = (
            jnp.where(has_bg, h_bg, 0.0)
            + jnp.where(has_fg, h_fg, 0.0)
            + jnp.where(both, 300.0 / (1.0 + dist), 0.0)
        )
        valid = (
            has_bg.astype(jnp.float32)
            + has_fg.astype(jnp.float32)
            + both.astype(jnp.float32)
        )
        valid_den = jnp.where(valid > 0.0, valid, 1.0)
        per_b = jnp.where(valid > 0.0, loss / valid_den, 0.0)
        o_ref[...] = jnp.sum(per_b, keepdims=True) / B

    return pl.pallas_call(
        ck,
        out_shape=jax.ShapeDtypeStruct((1, 1), jnp.float32),
        interpret=interpret,
    )(partials)


def kernel(prediction, target):
    predf = prediction.astype(jnp.float32)
    tgtf = target.astype(jnp.float32)
    partials = _sc_partials(predf, tgtf)
    partials = partials.reshape(NW, B, NQ, L).transpose(1, 2, 0, 3)
    partials = partials.reshape(B, NQ, NW * L)
    total = _combine(partials)
    return total[0, 0]
